# Initial kernel scaffold; baseline (speedup 1.0000x reference)
import jax
import jax.numpy as jnp
from jax.experimental import pallas as pl


def kernel(x, edge_index, edge_attr, batch, Wq1, bq1, Wk1, bk1, Wv1, bv1, We1, Wskip1, bskip1, Wq2, bq2, Wk2, bk2, Wv2, bv2, We2, Wskip2, bskip2, Wf1, bf1, Wf2, bf2, Wf3, bf3, Wf4, bf4):
    raise NotImplementedError("write your pallas kernel here")


# speedup vs baseline: 4.1017x; 4.1017x over previous
import math

import jax
import jax.numpy as jnp
from jax import lax
from jax.experimental import pallas as pl
from jax.experimental.pallas import tpu as pltpu
from jax.experimental.pallas import tpu_sc as plsc

_H = 4
_N = 10000
_E = 320000
_NG = 64
_NC = 2
_NS = 16
_NW = _NC * _NS
_EPT = _E // _NW
_B = 80
_NCHUNK = _EPT // _B


def _mm_call(x, wbs, row_block):
    n, din = x.shape
    nw = len(wbs)
    has_b = [b is not None for _, b in wbs]
    args = [x]
    in_specs = [pl.BlockSpec((row_block, din), lambda i: (i, 0))]
    for w, b in wbs:
        args.append(w)
        in_specs.append(pl.BlockSpec(w.shape, lambda i: (0, 0)))
        if b is not None:
            args.append(b)
            in_specs.append(pl.BlockSpec(b.shape, lambda i: (0, 0)))

    def body(*refs):
        xs = refs[0][...]
        outs = refs[len(refs) - nw:]
        r = 1
        for j in range(nw):
            w = refs[r][...]
            r += 1
            y = jnp.dot(xs, w, preferred_element_type=jnp.float32)
            if has_b[j]:
                y = y + refs[r][...]
                r += 1
            outs[j][...] = y

    return pl.pallas_call(
        body,
        grid=(n // row_block,),
        in_specs=in_specs,
        out_specs=[pl.BlockSpec((row_block, w.shape[1]), lambda i: (i, 0))
                   for w, _ in wbs],
        out_shape=[jax.ShapeDtypeStruct((n, w.shape[1]), jnp.float32)
                   for w, _ in wbs],
    )(*args)


def _combine(agg, skip):
    n, ch = skip.shape
    rb = 2000

    def body(a_ref, s_ref, o_ref):
        a = a_ref[...]
        o_ref[...] = jnp.maximum(a[0] + a[1] + s_ref[...], 0.0)

    return pl.pallas_call(
        body,
        grid=(n // rb,),
        in_specs=[pl.BlockSpec((2, rb, ch), lambda i: (0, i, 0)),
                  pl.BlockSpec((rb, ch), lambda i: (i, 0))],
        out_specs=pl.BlockSpec((rb, ch), lambda i: (i, 0)),
        out_shape=jax.ShapeDtypeStruct((n, ch), jnp.float32),
    )(agg, skip)


def _sum_parts(parts):
    hn = _H * _N

    def body(p_ref, o_ref):
        a = p_ref[...]
        acc = a[0:1]
        for t in range(1, _NW):
            acc = acc + a[t:t + 1]
        o_ref[...] = acc

    return pl.pallas_call(
        body,
        in_specs=[pl.BlockSpec((_NW, hn), lambda: (0, 0))],
        out_specs=pl.BlockSpec((1, hn), lambda: (0, 0)),
        out_shape=jax.ShapeDtypeStruct((1, hn), jnp.float32),
        grid=(),
    )(parts)


def _head(agg2, skip2, batch_row, w1, b1, w2, b2, w3, b3, w4, b4):
    def body(a_ref, sk_ref, bt_ref, w1r, b1r, w2r, b2r, w3r, b3r, w4r, b4r,
             o_ref):
        a = a_ref[...]
        h = jnp.maximum(a[0] + a[1] + sk_ref[...], 0.0)
        gid = lax.broadcasted_iota(jnp.int32, (_NG, 1), 0)
        mask = (gid == bt_ref[...]).astype(jnp.float32)
        sums = jnp.dot(mask, h, preferred_element_type=jnp.float32)
        cnt = jnp.sum(mask, axis=1, keepdims=True)
        g = sums / jnp.maximum(cnt, 1.0)
        z = jnp.maximum(jnp.dot(g, w1r[...],
                                preferred_element_type=jnp.float32) + b1r[...], 0.0)
        z = jnp.maximum(jnp.dot(z, w2r[...],
                                preferred_element_type=jnp.float32) + b2r[...], 0.0)
        z = jnp.maximum(jnp.dot(z, w3r[...],
                                preferred_element_type=jnp.float32) + b3r[...], 0.0)
        o_ref[...] = jnp.dot(z, w4r[...],
                             preferred_element_type=jnp.float32) + b4r[...]

    full = lambda s: pl.BlockSpec(s, lambda: tuple(0 for _ in s))
    args = (agg2, skip2, batch_row, w1, b1, w2, b2, w3, b3, w4, b4)
    return pl.pallas_call(
        body,
        in_specs=[full(a.shape) for a in args],
        out_specs=full((_NG, 10)),
        out_shape=jax.ShapeDtypeStruct((_NG, 10), jnp.float32),
        grid=(),
    )(*args)


def _sc_pass1(D, CH):
    inv = 1.0 / math.sqrt(float(CH))
    mesh = plsc.VectorSubcoreMesh(core_axis_name="c", subcore_axis_name="s")

    def body(q_hbm, k_hbm, e_hbm, src_hbm, dst_hbm, p_hbm, s_hbm,
             src_v, dst_v, qrows, krows, erows, pbuf, s_loc, sem):
        cid = lax.axis_index("c")
        sid = lax.axis_index("s")
        wid = sid * _NC + cid
        iota16 = lax.iota(jnp.int32, 16)
        zero16 = jnp.zeros((16,), jnp.float32)

        def zbody(i, carry):
            plsc.store_scatter(s_loc, [i * 16 + iota16], zero16)
            return carry
        lax.fori_loop(0, (_H * _N) // 16, zbody, 0)

        def chunk(ci, carry):
            base = wid * _EPT + ci * _B
            pltpu.sync_copy(src_hbm.at[pl.ds(base, _B)], src_v)
            pltpu.sync_copy(dst_hbm.at[pl.ds(base, _B)], dst_v)
            pltpu.async_copy(q_hbm.at[dst_v], qrows, sem).wait()
            pltpu.async_copy(k_hbm.at[src_v], krows, sem).wait()
            pltpu.sync_copy(e_hbm.at[pl.ds(base, _B)], erows)
            for g in range(_B // 16):
                eidx = iota16 + (g * 16)
                dst16 = dst_v[pl.ds(g * 16, 16)]
                for h in range(_H):
                    def dot_body(cc, acc, _h=h, _eidx=eidx):
                        col = jnp.full((16,), _h * CH, jnp.int32) + cc
                        qv = plsc.load_gather(qrows, [_eidx, col])
                        kv = (plsc.load_gather(krows, [_eidx, col])
                              + plsc.load_gather(erows, [_eidx, col]))
                        return acc + qv * kv
                    alpha = lax.fori_loop(0, CH, dot_body, zero16) * inv
                    p16 = jnp.exp(alpha)
                    pbuf[pl.ds(h * _B + g * 16, 16)] = p16
                    plsc.addupdate_scatter(
                        s_loc, [jnp.full((16,), h * _N, jnp.int32) + dst16],
                        p16)
            for h in range(_H):
                pltpu.sync_copy(pbuf.at[pl.ds(h * _B, _B)],
                                p_hbm.at[pl.ds(h * _E + base, _B)])
            return carry
        lax.fori_loop(0, _NCHUNK, chunk, 0)
        pltpu.sync_copy(s_loc, s_hbm.at[pl.ds(wid * _H * _N, _H * _N)])

    return pl.kernel(
        body,
        compiler_params=pltpu.CompilerParams(
            needs_layout_passes=False, use_tc_tiling_on_sc=False),
        out_type=[jax.ShapeDtypeStruct((_H * _E,), jnp.float32),
                  jax.ShapeDtypeStruct((_NW * _H * _N,), jnp.float32)],
        mesh=mesh,
        scratch_types=[
            pltpu.VMEM((_B,), jnp.int32),
            pltpu.VMEM((_B,), jnp.int32),
            pltpu.VMEM((_B, D), jnp.float32),
            pltpu.VMEM((_B, D), jnp.float32),
            pltpu.VMEM((_B, D), jnp.float32),
            pltpu.VMEM((_H * _B,), jnp.float32),
            pltpu.VMEM((_H * _N,), jnp.float32),
            pltpu.SemaphoreType.DMA,
        ],
    )


def _sc_pass2(D, CH):
    mesh = plsc.VectorSubcoreMesh(core_axis_name="c", subcore_axis_name="s")

    def body(v_hbm, e_hbm, p_hbm, s_hbm, z_hbm, src_hbm, dst_hbm, agg_hbm,
             src_v, dst_v, vrows, erows, pbuf, contrib, s_loc,
             agg_sh, sem):
        cid = lax.axis_index("c")
        sid = lax.axis_index("s")
        wid = sid * _NC + cid
        iota16 = lax.iota(jnp.int32, 16)

        pltpu.sync_copy(s_hbm, s_loc)

        @pl.when(sid == 0)
        def _():
            pltpu.sync_copy(z_hbm, agg_sh)
        plsc.subcore_barrier()

        def chunk(ci, carry):
            base = wid * _EPT + ci * _B
            pltpu.sync_copy(src_hbm.at[pl.ds(base, _B)], src_v)
            pltpu.sync_copy(dst_hbm.at[pl.ds(base, _B)], dst_v)
            pltpu.async_copy(v_hbm.at[src_v], vrows, sem).wait()
            pltpu.sync_copy(e_hbm.at[pl.ds(base, _B)], erows)
            for h in range(_H):
                pltpu.sync_copy(p_hbm.at[pl.ds(h * _E + base, _B)],
                                pbuf.at[pl.ds(h * _B, _B)])
            for g in range(_B // 16):
                eidx = iota16 + (g * 16)
                dst16 = dst_v[pl.ds(g * 16, 16)]
                ws = []
                for h in range(_H):
                    s16 = plsc.load_gather(
                        s_loc, [jnp.full((16,), h * _N, jnp.int32) + dst16])
                    p16 = pbuf[pl.ds(h * _B + g * 16, 16)]
                    ws.append(p16 * (1.0 / _H) / (s16 + 1e-16))

                def cc_body(cc, carry2, _ws=ws, _eidx=eidx):
                    colb = jnp.full((16,), 0, jnp.int32) + cc
                    acc = jnp.zeros((16,), jnp.float32)
                    for h in range(_H):
                        col = colb + h * CH
                        vv = (plsc.load_gather(vrows, [_eidx, col])
                              + plsc.load_gather(erows, [_eidx, col]))
                        acc = acc + _ws[h] * vv
                    plsc.store_scatter(contrib, [_eidx, colb], acc)
                    return carry2
                lax.fori_loop(0, CH, cc_body, 0)
            pltpu.sync_copy(contrib, agg_sh.at[dst_v], add=True)
            return carry
        lax.fori_loop(0, _NCHUNK, chunk, 0)
        plsc.subcore_barrier()

        @pl.when(sid == 0)
        def _():
            pltpu.sync_copy(agg_sh, agg_hbm.at[cid])

    return pl.kernel(
        body,
        compiler_params=pltpu.CompilerParams(
            needs_layout_passes=False, use_tc_tiling_on_sc=False),
        out_type=jax.ShapeDtypeStruct((_NC, _N, CH), jnp.float32),
        mesh=mesh,
        scratch_types=[
            pltpu.VMEM((_B,), jnp.int32),
            pltpu.VMEM((_B,), jnp.int32),
            pltpu.VMEM((_B, D), jnp.float32),
            pltpu.VMEM((_B, D), jnp.float32),
            pltpu.VMEM((_H * _B,), jnp.float32),
            pltpu.VMEM((_B, CH), jnp.float32),
            pltpu.VMEM((_H * _N,), jnp.float32),
            pltpu.VMEM_SHARED((_N, CH), jnp.float32),
            pltpu.SemaphoreType.DMA,
        ],
    )


def kernel(x, edge_index, edge_attr, batch,
           Wq1, bq1, Wk1, bk1, Wv1, bv1, We1, Wskip1, bskip1,
           Wq2, bq2, Wk2, bk2, Wv2, bv2, We2, Wskip2, bskip2,
           Wf1, bf1, Wf2, bf2, Wf3, bf3, Wf4, bf4):
    ei = jnp.asarray(edge_index, jnp.int32)
    src = ei[0]
    dst = ei[1]
    row = lambda b: b.reshape(1, -1)

    q1, k1, v1, skip1 = _mm_call(
        x, [(Wq1, row(bq1)), (Wk1, row(bk1)), (Wv1, row(bv1)),
            (Wskip1, row(bskip1))], 2000)
    e1 = _mm_call(edge_attr, [(We1, None)], 4000)[0]
    p1, s_parts1 = _sc_pass1(128, 32)(q1, k1, e1, src, dst)
    s1 = _sum_parts(s_parts1.reshape(_NW, _H * _N)).reshape(_H * _N)
    agg1 = _sc_pass2(128, 32)(
        v1, e1, p1, s1, jnp.zeros((_N, 32), jnp.float32), src, dst)
    h1 = _combine(agg1, skip1)

    q2, k2, v2, skip2 = _mm_call(
        h1, [(Wq2, row(bq2)), (Wk2, row(bk2)), (Wv2, row(bv2)),
             (Wskip2, row(bskip2))], 2000)
    e2 = _mm_call(edge_attr, [(We2, None)], 4000)[0]
    p2, s_parts2 = _sc_pass1(256, 64)(q2, k2, e2, src, dst)
    s2 = _sum_parts(s_parts2.reshape(_NW, _H * _N)).reshape(_H * _N)
    agg2 = _sc_pass2(256, 64)(
        v2, e2, p2, s2, jnp.zeros((_N, 64), jnp.float32), src, dst)

    return _head(agg2, skip2, row(jnp.asarray(batch, jnp.int32)),
                 Wf1, row(bf1), Wf2, row(bf2), Wf3, row(bf3), Wf4, row(bf4))


# speedup vs baseline: 4.5714x; 1.1145x over previous
import math

import jax
import jax.numpy as jnp
from jax import lax
from jax.experimental import pallas as pl
from jax.experimental.pallas import tpu as pltpu
from jax.experimental.pallas import tpu_sc as plsc

_H = 4
_N = 10000
_E = 320000
_NG = 64
_NC = 2
_NS = 16
_NW = _NC * _NS
_EPT = _E // _NW
_B = 80
_NCHUNK = _EPT // _B


def _mm_call(x, wbs, row_block):
    n, din = x.shape
    nw = len(wbs)
    has_b = [b is not None for _, b in wbs]
    args = [x]
    in_specs = [pl.BlockSpec((row_block, din), lambda i: (i, 0))]
    for w, b in wbs:
        args.append(w)
        in_specs.append(pl.BlockSpec(w.shape, lambda i: (0, 0)))
        if b is not None:
            args.append(b)
            in_specs.append(pl.BlockSpec(b.shape, lambda i: (0, 0)))

    def body(*refs):
        xs = refs[0][...]
        outs = refs[len(refs) - nw:]
        r = 1
        for j in range(nw):
            w = refs[r][...]
            r += 1
            y = jnp.dot(xs, w, preferred_element_type=jnp.float32)
            if has_b[j]:
                y = y + refs[r][...]
                r += 1
            outs[j][...] = y

    return pl.pallas_call(
        body,
        grid=(n // row_block,),
        in_specs=in_specs,
        out_specs=[pl.BlockSpec((row_block, w.shape[1]), lambda i: (i, 0))
                   for w, _ in wbs],
        out_shape=[jax.ShapeDtypeStruct((n, w.shape[1]), jnp.float32)
                   for w, _ in wbs],
    )(*args)


def _combine(agg, skip):
    n, ch = skip.shape
    rb = 2000

    def body(a_ref, s_ref, o_ref):
        a = a_ref[...]
        o_ref[...] = jnp.maximum(a[0] + a[1] + s_ref[...], 0.0)

    return pl.pallas_call(
        body,
        grid=(n // rb,),
        in_specs=[pl.BlockSpec((2, rb, ch), lambda i: (0, i, 0)),
                  pl.BlockSpec((rb, ch), lambda i: (i, 0))],
        out_specs=pl.BlockSpec((rb, ch), lambda i: (i, 0)),
        out_shape=jax.ShapeDtypeStruct((n, ch), jnp.float32),
    )(agg, skip)


def _sum_parts(parts):
    hn = _H * _N

    def body(p_ref, o_ref):
        a = p_ref[...]
        acc = a[0:1]
        for t in range(1, _NW):
            acc = acc + a[t:t + 1]
        o_ref[...] = acc

    return pl.pallas_call(
        body,
        in_specs=[pl.BlockSpec((_NW, hn), lambda: (0, 0))],
        out_specs=pl.BlockSpec((1, hn), lambda: (0, 0)),
        out_shape=jax.ShapeDtypeStruct((1, hn), jnp.float32),
        grid=(),
    )(parts)


def _head(agg2, skip2, batch_row, w1, b1, w2, b2, w3, b3, w4, b4):
    def body(a_ref, sk_ref, bt_ref, w1r, b1r, w2r, b2r, w3r, b3r, w4r, b4r,
             o_ref):
        a = a_ref[...]
        h = jnp.maximum(a[0] + a[1] + sk_ref[...], 0.0)
        gid = lax.broadcasted_iota(jnp.int32, (_NG, 1), 0)
        mask = (gid == bt_ref[...]).astype(jnp.float32)
        sums = jnp.dot(mask, h, preferred_element_type=jnp.float32)
        cnt = jnp.sum(mask, axis=1, keepdims=True)
        g = sums / jnp.maximum(cnt, 1.0)
        z = jnp.maximum(jnp.dot(g, w1r[...],
                                preferred_element_type=jnp.float32) + b1r[...], 0.0)
        z = jnp.maximum(jnp.dot(z, w2r[...],
                                preferred_element_type=jnp.float32) + b2r[...], 0.0)
        z = jnp.maximum(jnp.dot(z, w3r[...],
                                preferred_element_type=jnp.float32) + b3r[...], 0.0)
        o_ref[...] = jnp.dot(z, w4r[...],
                             preferred_element_type=jnp.float32) + b4r[...]

    full = lambda s: pl.BlockSpec(s, lambda: tuple(0 for _ in s))
    args = (agg2, skip2, batch_row, w1, b1, w2, b2, w3, b3, w4, b4)
    return pl.pallas_call(
        body,
        in_specs=[full(a.shape) for a in args],
        out_specs=full((_NG, 10)),
        out_shape=jax.ShapeDtypeStruct((_NG, 10), jnp.float32),
        grid=(),
    )(*args)


def _sc_pass1(D, CH):
    inv = 1.0 / math.sqrt(float(CH))
    mesh = plsc.VectorSubcoreMesh(core_axis_name="c", subcore_axis_name="s")

    def body(q_hbm, k_hbm, e_hbm, src_hbm, dst_hbm, p_hbm, s_hbm,
             src_v, dst_v, qrows, krows, erows, pbuf, s_loc, sem, sem2, sem3):
        cid = lax.axis_index("c")
        sid = lax.axis_index("s")
        wid = sid * _NC + cid
        iota16 = lax.iota(jnp.int32, 16)
        zero16 = jnp.zeros((16,), jnp.float32)

        def zbody(i, carry):
            plsc.store_scatter(s_loc, [i * 16 + iota16], zero16)
            return carry
        lax.fori_loop(0, (_H * _N) // 16, zbody, 0)

        def chunk(ci, carry):
            base = wid * _EPT + ci * _B
            pltpu.sync_copy(src_hbm.at[pl.ds(base, _B)], src_v)
            pltpu.sync_copy(dst_hbm.at[pl.ds(base, _B)], dst_v)
            cq = pltpu.async_copy(q_hbm.at[dst_v], qrows, sem)
            ck = pltpu.async_copy(k_hbm.at[src_v], krows, sem2)
            ce = pltpu.async_copy(e_hbm.at[pl.ds(base, _B)], erows, sem3)
            cq.wait()
            ck.wait()
            ce.wait()
            for g in range(_B // 16):
                eidx = iota16 + (g * 16)
                dst16 = dst_v[pl.ds(g * 16, 16)]
                for h in range(_H):
                    def dot_body(cc, acc, _h=h, _eidx=eidx):
                        for j in range(4):
                            col = (jnp.full((16,), _h * CH, jnp.int32)
                                   + (cc * 4 + j))
                            qv = plsc.load_gather(qrows, [_eidx, col])
                            kv = (plsc.load_gather(krows, [_eidx, col])
                                  + plsc.load_gather(erows, [_eidx, col]))
                            acc = acc + qv * kv
                        return acc
                    alpha = lax.fori_loop(0, CH // 4, dot_body, zero16) * inv
                    p16 = jnp.exp(alpha)
                    plsc.store_scatter(pbuf, [eidx * _H + h], p16)
                    plsc.addupdate_scatter(
                        s_loc, [jnp.full((16,), h * _N, jnp.int32) + dst16],
                        p16)
            pltpu.sync_copy(pbuf, p_hbm.at[pl.ds(base * _H, _B * _H)])
            return carry
        lax.fori_loop(0, _NCHUNK, chunk, 0)
        pltpu.sync_copy(s_loc, s_hbm.at[pl.ds(wid * _H * _N, _H * _N)])

    return pl.kernel(
        body,
        compiler_params=pltpu.CompilerParams(
            needs_layout_passes=False, use_tc_tiling_on_sc=False),
        out_type=[jax.ShapeDtypeStruct((_H * _E,), jnp.float32),
                  jax.ShapeDtypeStruct((_NW * _H * _N,), jnp.float32)],
        mesh=mesh,
        scratch_types=[
            pltpu.VMEM((_B,), jnp.int32),
            pltpu.VMEM((_B,), jnp.int32),
            pltpu.VMEM((_B, D), jnp.float32),
            pltpu.VMEM((_B, D), jnp.float32),
            pltpu.VMEM((_B, D), jnp.float32),
            pltpu.VMEM((_H * _B,), jnp.float32),
            pltpu.VMEM((_H * _N,), jnp.float32),
            pltpu.SemaphoreType.DMA,
            pltpu.SemaphoreType.DMA,
            pltpu.SemaphoreType.DMA,
        ],
    )


def _sc_pass2(D, CH):
    mesh = plsc.VectorSubcoreMesh(core_axis_name="c", subcore_axis_name="s")

    def body(v_hbm, e_hbm, p_hbm, s_hbm, z_hbm, src_hbm, dst_hbm, agg_hbm,
             src_v, dst_v, vrows, erows, pbuf, contrib, s_loc,
             agg_sh, sem, sem2, sem3):
        cid = lax.axis_index("c")
        sid = lax.axis_index("s")
        wid = sid * _NC + cid
        iota16 = lax.iota(jnp.int32, 16)

        pltpu.sync_copy(s_hbm, s_loc)

        @pl.when(sid == 0)
        def _():
            pltpu.sync_copy(z_hbm, agg_sh)
        plsc.subcore_barrier()

        def chunk(ci, carry):
            base = wid * _EPT + ci * _B
            pltpu.sync_copy(src_hbm.at[pl.ds(base, _B)], src_v)
            pltpu.sync_copy(dst_hbm.at[pl.ds(base, _B)], dst_v)
            cv = pltpu.async_copy(v_hbm.at[src_v], vrows, sem)
            ce = pltpu.async_copy(e_hbm.at[pl.ds(base, _B)], erows, sem2)
            cp = pltpu.async_copy(p_hbm.at[pl.ds(base * _H, _B * _H)],
                                  pbuf, sem3)
            cv.wait()
            ce.wait()
            cp.wait()
            for g in range(_B // 16):
                eidx = iota16 + (g * 16)
                dst16 = dst_v[pl.ds(g * 16, 16)]
                ws = []
                for h in range(_H):
                    s16 = plsc.load_gather(
                        s_loc, [jnp.full((16,), h * _N, jnp.int32) + dst16])
                    p16 = plsc.load_gather(pbuf, [eidx * _H + h])
                    ws.append(p16 * (1.0 / _H) / (s16 + 1e-16))

                def cc_body(cc, carry2, _ws=ws, _eidx=eidx):
                    for j in range(4):
                        colb = jnp.full((16,), 0, jnp.int32) + (cc * 4 + j)
                        acc = jnp.zeros((16,), jnp.float32)
                        for h in range(_H):
                            col = colb + h * CH
                            vv = (plsc.load_gather(vrows, [_eidx, col])
                                  + plsc.load_gather(erows, [_eidx, col]))
                            acc = acc + _ws[h] * vv
                        plsc.store_scatter(contrib, [_eidx, colb], acc)
                    return carry2
                lax.fori_loop(0, CH // 4, cc_body, 0)
            pltpu.sync_copy(contrib, agg_sh.at[dst_v], add=True)
            return carry
        lax.fori_loop(0, _NCHUNK, chunk, 0)
        plsc.subcore_barrier()

        @pl.when(sid == 0)
        def _():
            pltpu.sync_copy(agg_sh, agg_hbm.at[cid])

    return pl.kernel(
        body,
        compiler_params=pltpu.CompilerParams(
            needs_layout_passes=False, use_tc_tiling_on_sc=False),
        out_type=jax.ShapeDtypeStruct((_NC, _N, CH), jnp.float32),
        mesh=mesh,
        scratch_types=[
            pltpu.VMEM((_B,), jnp.int32),
            pltpu.VMEM((_B,), jnp.int32),
            pltpu.VMEM((_B, D), jnp.float32),
            pltpu.VMEM((_B, D), jnp.float32),
            pltpu.VMEM((_H * _B,), jnp.float32),
            pltpu.VMEM((_B, CH), jnp.float32),
            pltpu.VMEM((_H * _N,), jnp.float32),
            pltpu.VMEM_SHARED((_N, CH), jnp.float32),
            pltpu.SemaphoreType.DMA,
            pltpu.SemaphoreType.DMA,
            pltpu.SemaphoreType.DMA,
        ],
    )


def kernel(x, edge_index, edge_attr, batch,
           Wq1, bq1, Wk1, bk1, Wv1, bv1, We1, Wskip1, bskip1,
           Wq2, bq2, Wk2, bk2, Wv2, bv2, We2, Wskip2, bskip2,
           Wf1, bf1, Wf2, bf2, Wf3, bf3, Wf4, bf4):
    ei = jnp.asarray(edge_index, jnp.int32)
    src = ei[0]
    dst = ei[1]
    row = lambda b: b.reshape(1, -1)

    q1, k1, v1, skip1 = _mm_call(
        x, [(Wq1, row(bq1)), (Wk1, row(bk1)), (Wv1, row(bv1)),
            (Wskip1, row(bskip1))], 2000)
    e1 = _mm_call(edge_attr, [(We1, None)], 4000)[0]
    p1, s_parts1 = _sc_pass1(128, 32)(q1, k1, e1, src, dst)
    s1 = _sum_parts(s_parts1.reshape(_NW, _H * _N)).reshape(_H * _N)
    agg1 = _sc_pass2(128, 32)(
        v1, e1, p1, s1, jnp.zeros((_N, 32), jnp.float32), src, dst)
    h1 = _combine(agg1, skip1)

    q2, k2, v2, skip2 = _mm_call(
        h1, [(Wq2, row(bq2)), (Wk2, row(bk2)), (Wv2, row(bv2)),
             (Wskip2, row(bskip2))], 2000)
    e2 = _mm_call(edge_attr, [(We2, None)], 4000)[0]
    p2, s_parts2 = _sc_pass1(256, 64)(q2, k2, e2, src, dst)
    s2 = _sum_parts(s_parts2.reshape(_NW, _H * _N)).reshape(_H * _N)
    agg2 = _sc_pass2(256, 64)(
        v2, e2, p2, s2, jnp.zeros((_N, 64), jnp.float32), src, dst)

    return _head(agg2, skip2, row(jnp.asarray(batch, jnp.int32)),
                 Wf1, row(bf1), Wf2, row(bf2), Wf3, row(bf3), Wf4, row(bf4))
